# baseline (speedup 1.0000x reference)
import jax
import jax.numpy as jnp
from jax import lax
from jax.experimental import pallas as pl
from jax.experimental.pallas import tpu as pltpu
from jax.experimental.pallas import tpu_sc as plsc

N_TOK = 16384
D = 128
C0, C1, C2, C3 = 10000, 60000, 190000, 1000000
NW = 32
B_W = N_TOK // NW
G = 4
B_G = B_W // G


def _sc_body(inp, hemb, e1, e2v, e3v, h_out, t1_out, t2_out, t3_out,
             idx_v, ih_v, i1_v, i2_v, i3_v, hbuf, t1buf, t2buf, t3buf, sem):
    wid = lax.axis_index("s") * 2 + lax.axis_index("c")
    base = wid * B_W
    pltpu.sync_copy(inp.at[pl.ds(base, B_W)], idx_v)
    for i in range(B_W // 16):
        v = idx_v[pl.ds(i * 16, 16)]
        j, o = i // 8, (i % 8) * 16
        ih_v[j, pl.ds(o, 16)] = jnp.clip(v, 0, C0 - 1)
        i1_v[j, pl.ds(o, 16)] = jnp.clip(v - C0, 0, C1 - C0 - 1)
        i2_v[j, pl.ds(o, 16)] = lax.shift_right_logical(
            jnp.clip(v - C1, 0, C2 - C1 - 1), 1)
        i3_v[j, pl.ds(o, 16)] = lax.shift_right_logical(
            jnp.clip(v - C2, 0, C3 - C2 - 1), 3)
    copies = []
    for j in range(G):
        s = pl.ds(j * B_G, B_G)
        copies.append(pltpu.make_async_copy(hemb.at[ih_v.at[j]], hbuf.at[s], sem))
        copies.append(pltpu.make_async_copy(e1.at[i1_v.at[j]], t1buf.at[s], sem))
        copies.append(pltpu.make_async_copy(e2v.at[i2_v.at[j]], t2buf.at[s], sem))
        copies.append(pltpu.make_async_copy(e3v.at[i3_v.at[j]], t3buf.at[s], sem))
    for c in copies:
        c.start()
    for c in copies:
        c.wait()
    out_s = pl.ds(base, B_W)
    pltpu.sync_copy(hbuf, h_out.at[out_s])
    pltpu.sync_copy(t1buf, t1_out.at[out_s])
    pltpu.sync_copy(t2buf, t2_out.at[out_s])
    pltpu.sync_copy(t3buf, t3_out.at[out_s])


@jax.jit
def _sc_gather(inp, hemb, e1, e2v, e3v):
    f32 = jnp.float32
    return pl.kernel(
        _sc_body,
        out_type=(
            jax.ShapeDtypeStruct((N_TOK, 128), f32),
            jax.ShapeDtypeStruct((N_TOK, 32), f32),
            jax.ShapeDtypeStruct((N_TOK, 16), f32),
            jax.ShapeDtypeStruct((N_TOK, 16), f32),
        ),
        mesh=plsc.VectorSubcoreMesh(core_axis_name="c", subcore_axis_name="s"),
        compiler_params=pltpu.CompilerParams(use_tc_tiling_on_sc=False),
        scratch_types=[
            pltpu.VMEM((B_W,), jnp.int32),
            pltpu.VMEM((G, B_G), jnp.int32),
            pltpu.VMEM((G, B_G), jnp.int32),
            pltpu.VMEM((G, B_G), jnp.int32),
            pltpu.VMEM((G, B_G), jnp.int32),
            pltpu.VMEM((B_W, 128), f32),
            pltpu.VMEM((B_W, 32), f32),
            pltpu.VMEM((B_W, 16), f32),
            pltpu.VMEM((B_W, 16), f32),
            pltpu.SemaphoreType.DMA,
        ],
    )(inp, hemb, e1, e2v, e3v)


B_M = 1024


def _mm_body(inp, h, t1, t2, t3, hw, w1, w2d, w3d, out):
    v = inp[...]
    lane = lax.broadcasted_iota(jnp.int32, (B_M, 16), 1)
    hm = jnp.where(v < C0, h[...], 0.0)
    t1m = jnp.where((v >= C0) & (v < C1), t1[...], 0.0)
    t2m = jnp.where((v >= C1) & (v < C2) & ((lane >> 3) == ((v - C1) & 1)),
                    t2[...], 0.0)
    t3m = jnp.where((v >= C2) & ((lane >> 1) == ((v - C2) & 7)), t3[...], 0.0)
    acc = jnp.dot(hm, hw[...], preferred_element_type=jnp.float32)
    acc += jnp.dot(t1m, w1[...], preferred_element_type=jnp.float32)
    acc += jnp.dot(t2m, w2d[...], preferred_element_type=jnp.float32)
    acc += jnp.dot(t3m, w3d[...], preferred_element_type=jnp.float32)
    out[...] = acc


@jax.jit
def _tc_project(inp2, h, t1, t2, t3, hw, w1, w2d, w3d):
    nb = N_TOK // B_M
    blk = lambda r: pl.BlockSpec((B_M, r), lambda b: (b, 0))
    full = lambda a, b: pl.BlockSpec((a, b), lambda _: (0, 0))
    return pl.pallas_call(
        _mm_body,
        grid=(nb,),
        in_specs=[blk(1), blk(128), blk(32), blk(16), blk(16),
                  full(128, 128), full(32, 128), full(16, 128), full(16, 128)],
        out_specs=blk(128),
        out_shape=jax.ShapeDtypeStruct((N_TOK, D), jnp.float32),
    )(inp2, h, t1, t2, t3, hw, w1, w2d, w3d)


def kernel(input, head_emb, head_W, emb1, W1, emb2, W2, emb3, W3):
    e2v = emb2.reshape((C2 - C1) // 2, 16)
    e3v = emb3.reshape((C3 - C2) // 8, 16)
    h, t1, t2, t3 = _sc_gather(input, head_emb, emb1, e2v, e3v)
    w2d = jnp.concatenate([W2, W2], axis=0)
    w3d = jnp.tile(W3, (8, 1))
    return _tc_project(input.reshape(N_TOK, 1), h, t1, t2, t3,
                       head_W, W1, w2d, w3d)


# speedup vs baseline: 1.3610x; 1.3610x over previous
import jax
import jax.numpy as jnp
from jax import lax
from jax.experimental import pallas as pl
from jax.experimental.pallas import tpu as pltpu
from jax.experimental.pallas import tpu_sc as plsc

N_TOK = 16384
D = 128
C0, C1, C2, C3 = 10000, 60000, 190000, 1000000
WB1 = C0
WB2 = WB1 + (C1 - C0) * 32 // 128
WB3 = WB2 + (C2 - C1) * 8 // 128
N_WIN = WB3 + ((C3 - C2) * 2 + 127) // 128
PAD = N_WIN * 128 - (C0 * 128 + (C1 - C0) * 32 + (C2 - C1) * 8 + (C3 - C2) * 2)

NW = 32
B_W = N_TOK // NW
G = 4
B_G = B_W // G


def _sc_body(inp, flat, out, idx_v, win_v, buf, sem):
    wid = lax.axis_index("s") * 2 + lax.axis_index("c")
    base = wid * B_W
    pltpu.sync_copy(inp.at[pl.ds(base, B_W)], idx_v)
    for i in range(B_W // 16):
        v = idx_v[pl.ds(i * 16, 16)]
        w = jnp.where(
            v < C0, v,
            jnp.where(
                v < C1, WB1 + lax.shift_right_logical(v - C0, 2),
                jnp.where(
                    v < C2, WB2 + lax.shift_right_logical(v - C1, 4),
                    WB3 + lax.shift_right_logical(v - C2, 6))))
        win_v[i // 8, pl.ds((i % 8) * 16, 16)] = w
    copies = [
        pltpu.make_async_copy(flat.at[win_v.at[j]],
                              buf.at[pl.ds(j * B_G, B_G)], sem)
        for j in range(G)
    ]
    for c in copies:
        c.start()
    for c in copies:
        c.wait()
    pltpu.sync_copy(buf, out.at[pl.ds(base, B_W)])


@jax.jit
def _sc_gather(inp, flat):
    return pl.kernel(
        _sc_body,
        out_type=jax.ShapeDtypeStruct((N_TOK, 128), jnp.float32),
        mesh=plsc.VectorSubcoreMesh(core_axis_name="c", subcore_axis_name="s"),
        compiler_params=pltpu.CompilerParams(use_tc_tiling_on_sc=False),
        scratch_types=[
            pltpu.VMEM((B_W,), jnp.int32),
            pltpu.VMEM((G, B_G), jnp.int32),
            pltpu.VMEM((B_W, 128), jnp.float32),
            pltpu.SemaphoreType.DMA,
        ],
    )(inp, flat)


B_M = 1024


def _mm_body(inp, gw, hw, w1d, w2d, w3d, out):
    v = inp[...]
    g = gw[...]
    lane = lax.broadcasted_iota(jnp.int32, (B_M, 128), 1)
    hm = jnp.where(v < C0, g, 0.0)
    t1m = jnp.where((v >= C0) & (v < C1) & ((lane >> 5) == ((v - C0) & 3)),
                    g, 0.0)
    t2m = jnp.where((v >= C1) & (v < C2) & ((lane >> 3) == ((v - C1) & 15)),
                    g, 0.0)
    t3m = jnp.where((v >= C2) & ((lane >> 1) == ((v - C2) & 63)), g, 0.0)
    acc = jnp.dot(hm, hw[...], preferred_element_type=jnp.float32)
    acc += jnp.dot(t1m, w1d[...], preferred_element_type=jnp.float32)
    acc += jnp.dot(t2m, w2d[...], preferred_element_type=jnp.float32)
    acc += jnp.dot(t3m, w3d[...], preferred_element_type=jnp.float32)
    out[...] = acc


@jax.jit
def _tc_project(inp2, gw, hw, w1d, w2d, w3d):
    nb = N_TOK // B_M
    blk = lambda r: pl.BlockSpec((B_M, r), lambda b: (b, 0))
    full = lambda a, b: pl.BlockSpec((a, b), lambda _: (0, 0))
    return pl.pallas_call(
        _mm_body,
        grid=(nb,),
        in_specs=[blk(1), blk(128),
                  full(128, 128), full(128, 128), full(128, 128),
                  full(128, 128)],
        out_specs=blk(128),
        out_shape=jax.ShapeDtypeStruct((N_TOK, D), jnp.float32),
    )(inp2, gw, hw, w1d, w2d, w3d)


def kernel(input, head_emb, head_W, emb1, W1, emb2, W2, emb3, W3):
    flat = jnp.concatenate([
        head_emb.reshape(-1), emb1.reshape(-1), emb2.reshape(-1),
        emb3.reshape(-1), jnp.zeros((PAD,), jnp.float32),
    ]).reshape(N_WIN, 128)
    gw = _sc_gather(input, flat)
    w1d = jnp.tile(W1, (4, 1))
    w2d = jnp.tile(W2, (16, 1))
    w3d = jnp.tile(W3, (64, 1))
    return _tc_project(input.reshape(N_TOK, 1), gw,
                       head_W, w1d, w2d, w3d)
